# Initial kernel scaffold; baseline (speedup 1.0000x reference)
import jax
import jax.numpy as jnp
from jax.experimental import pallas as pl


def kernel(x_nctv, A, W, b, bn_gamma, bn_beta, bn_mean, bn_var, Wd, bd, dbn_gamma, dbn_beta, dbn_mean, dbn_var):
    raise NotImplementedError("write your pallas kernel here")


# speedup vs baseline: 2.4444x; 2.4444x over previous
import jax
import jax.numpy as jnp
from jax.experimental import pallas as pl
from jax.experimental.pallas import tpu as pltpu


def _make_fused_kernel(C, T, V, K):
    def _fused_kernel(x_ref, a_ref, w_ref, shift_ref, o_ref):
        x = x_ref[0]
        cat = jnp.dot(x, a_ref[...], preferred_element_type=jnp.float32)
        parts = [cat[:, 2 * V * k:2 * V * (k + 1)].reshape(C, T * V)
                 for k in range(K)]
        parts.append(x.reshape(C, T * V))
        big = jnp.concatenate(parts, axis=0)
        y = jnp.dot(w_ref[...], big, preferred_element_type=jnp.float32)
        o_ref[0] = jnp.maximum(y + shift_ref[...], 0.0)
    return _fused_kernel


@jax.jit
def kernel(x_nctv, A, W, b, bn_gamma, bn_beta, bn_mean, bn_var,
           Wd, bd, dbn_gamma, dbn_beta, dbn_mean, dbn_var, eps=1e-5):
    N, C, T, V = x_nctv.shape
    K, O, _ = W.shape

    An = A / (jnp.sqrt(jnp.sum(A * A, axis=1, keepdims=True)) + 1e-4)
    bn_scale = bn_gamma / jnp.sqrt(bn_var + eps)
    bn_shift = bn_beta - bn_mean * bn_scale
    d_scale = dbn_gamma / jnp.sqrt(dbn_var + eps)
    d_shift = dbn_beta - dbn_mean * d_scale

    W_fold = W * bn_scale[None, :, None]
    Wd_fold = Wd * d_scale[:, None]
    W_all = jnp.concatenate([W_fold[0], W_fold[1], W_fold[2], Wd_fold],
                            axis=1)
    shift = (bn_scale * jnp.sum(b, axis=0) + bn_shift
             + d_scale * bd + d_shift).reshape(O, 1)

    z = jnp.zeros((K, V, V), jnp.float32)
    A2 = jnp.concatenate([jnp.concatenate([An, z], axis=2),
                          jnp.concatenate([z, An], axis=2)], axis=1)
    A2cat = jnp.concatenate([A2[k] for k in range(K)], axis=1)

    x2 = x_nctv.reshape(N, C * T // 2, 2 * V)

    out = pl.pallas_call(
        _make_fused_kernel(C, T, V, K),
        out_shape=jax.ShapeDtypeStruct((N, O, T * V), jnp.float32),
        grid=(N,),
        in_specs=[
            pl.BlockSpec((1, C * T // 2, 2 * V), lambda n: (n, 0, 0)),
            pl.BlockSpec((2 * V, K * 2 * V), lambda n: (0, 0)),
            pl.BlockSpec((O, (K + 1) * C), lambda n: (0, 0)),
            pl.BlockSpec((O, 1), lambda n: (0, 0)),
        ],
        out_specs=pl.BlockSpec((1, O, T * V), lambda n: (n, 0, 0)),
        compiler_params=pltpu.CompilerParams(
            dimension_semantics=("parallel",)),
    )(x2, A2cat, W_all, shift)

    return out.reshape(N, O, T, V)
